# Initial kernel scaffold; baseline (speedup 1.0000x reference)
import jax
import jax.numpy as jnp
from jax.experimental import pallas as pl


def kernel(x, emb, sizes):
    raise NotImplementedError("write your pallas kernel here")


# speedup vs baseline: 1.6884x; 1.6884x over previous
import jax
import jax.numpy as jnp
from jax import lax
from jax.experimental import pallas as pl
from jax.experimental.pallas import tpu as pltpu

DIM = 1024
EMB_ROWS = 4096
R = 512


def _read_rows(emb_ref, off):
    q = pl.multiple_of((off // 8) * 8, 8)
    rrem = off - q
    big = emb_ref[pl.ds(q, R + 8), :]
    rolled = pltpu.roll(big, (R + 8 - rrem) % (R + 8), axis=0)
    return rolled[:R, :]


def _tc_body(starts_ref, seg_first_ref, n_extra_ref, x_ref, emb_ref, o_ref):
    b = pl.program_id(0)
    base = b * R
    s0 = seg_first_ref[b]
    riota = lax.broadcasted_iota(jnp.int32, (R, 1), 0)

    def piece(k, acc):
        s = s0 + k
        st = starts_ref[s]
        boundary = st - base
        cand = _read_rows(emb_ref, base - st + R)
        return jnp.where(riota >= boundary, cand, acc)

    acc = _read_rows(emb_ref, base - starts_ref[s0] + R)
    acc = lax.fori_loop(1, n_extra_ref[b] + 1, piece, acc)
    o_ref[:, :] = acc + x_ref[:, :]


def _tc_call(x, emb_p, starts, seg_first, n_extra, interpret=False):
    total = x.shape[0]
    nb = (total + R - 1) // R
    grid_spec = pltpu.PrefetchScalarGridSpec(
        num_scalar_prefetch=3,
        grid=(nb,),
        in_specs=[
            pl.BlockSpec((R, DIM), lambda b, *_: (b, 0)),
            pl.BlockSpec((EMB_ROWS + 2 * R + 8, DIM), lambda b, *_: (0, 0)),
        ],
        out_specs=pl.BlockSpec((R, DIM), lambda b, *_: (b, 0)),
    )
    return pl.pallas_call(
        _tc_body,
        grid_spec=grid_spec,
        out_shape=jax.ShapeDtypeStruct((total, DIM), jnp.float32),
        compiler_params=pltpu.CompilerParams(
            dimension_semantics=("arbitrary",)),
        interpret=interpret,
    )(starts, seg_first, n_extra, x, emb_p)


def kernel(x, emb, sizes):
    total = x.shape[0]
    nb = (total + R - 1) // R
    sizes = sizes.astype(jnp.int32)
    csum = jnp.cumsum(sizes)
    starts = (csum - sizes).astype(jnp.int32)
    bstart = jnp.arange(nb, dtype=jnp.int32) * R
    seg_first = jnp.searchsorted(csum, bstart, side="right").astype(jnp.int32)
    last_row = jnp.minimum(bstart + (R - 1), total - 1)
    seg_last = jnp.searchsorted(csum, last_row, side="right").astype(jnp.int32)
    n_extra = seg_last - seg_first
    emb_p = jnp.pad(emb[:EMB_ROWS], ((R, R + 8), (0, 0)))
    return _tc_call(x, emb_p, starts, seg_first, n_extra)


# speedup vs baseline: 2.4657x; 1.4603x over previous
import jax
import jax.numpy as jnp
from jax import lax
from jax.experimental import pallas as pl
from jax.experimental.pallas import tpu as pltpu

DIM = 1024
EMB_ROWS = 4096
R = 512


def _tc_body(starts_ref, seg_first_ref, n_extra_ref, x_ref, emb_ref, o_ref):
    b = pl.program_id(0)
    base = b * R
    s0 = seg_first_ref[b]

    off0 = base - starts_ref[s0] + R
    q0 = pl.multiple_of((off0 // 8) * 8, 8)

    def store_piece0(k):
        def f():
            big = emb_ref[pl.ds(q0, R + 8), :]
            o_ref[:, :] = big[k:k + R, :] + x_ref[:, :]
        return f

    lax.switch(off0 - q0, [store_piece0(k) for k in range(8)])

    riota = lax.broadcasted_iota(jnp.int32, (R, 1), 0)

    def piece(j, carry):
        st = starts_ref[s0 + j]
        boundary = st - base
        offj = base - st + R
        qj = pl.multiple_of((offj // 8) * 8, 8)

        def store_piecej(k):
            def f():
                big = emb_ref[pl.ds(qj, R + 8), :]
                o_ref[:, :] = jnp.where(
                    riota >= boundary,
                    big[k:k + R, :] + x_ref[:, :],
                    o_ref[:, :])
            return f

        lax.switch(offj - qj, [store_piecej(k) for k in range(8)])
        return carry

    lax.fori_loop(1, n_extra_ref[b] + 1, piece, 0)


def _tc_call(x, emb_p, starts, seg_first, n_extra, interpret=False):
    total = x.shape[0]
    nb = (total + R - 1) // R
    grid_spec = pltpu.PrefetchScalarGridSpec(
        num_scalar_prefetch=3,
        grid=(nb,),
        in_specs=[
            pl.BlockSpec((R, DIM), lambda b, *_: (b, 0)),
            pl.BlockSpec((EMB_ROWS + 2 * R + 8, DIM), lambda b, *_: (0, 0)),
        ],
        out_specs=pl.BlockSpec((R, DIM), lambda b, *_: (b, 0)),
    )
    return pl.pallas_call(
        _tc_body,
        grid_spec=grid_spec,
        out_shape=jax.ShapeDtypeStruct((total, DIM), jnp.float32),
        compiler_params=pltpu.CompilerParams(
            dimension_semantics=("arbitrary",)),
        interpret=interpret,
    )(starts, seg_first, n_extra, x, emb_p)


def kernel(x, emb, sizes):
    total = x.shape[0]
    nb = (total + R - 1) // R
    sizes = sizes.astype(jnp.int32)
    csum = jnp.cumsum(sizes)
    starts = (csum - sizes).astype(jnp.int32)
    bstart = jnp.arange(nb, dtype=jnp.int32) * R
    seg_first = jnp.searchsorted(csum, bstart, side="right").astype(jnp.int32)
    last_row = jnp.minimum(bstart + (R - 1), total - 1)
    seg_last = jnp.searchsorted(csum, last_row, side="right").astype(jnp.int32)
    n_extra = seg_last - seg_first
    emb_p = jnp.pad(emb[:EMB_ROWS], ((R, R + 8), (0, 0)))
    return _tc_call(x, emb_p, starts, seg_first, n_extra)


# speedup vs baseline: 2.7372x; 1.1101x over previous
import jax
import jax.numpy as jnp
from jax import lax
from jax.experimental import pallas as pl
from jax.experimental.pallas import tpu as pltpu

DIM = 1024
EMB_ROWS = 4096
R = 512
S_ROWS = 2 * R + EMB_ROWS + 8


def _tc_body(starts_ref, seg_first_ref, n_extra_ref, x_ref, emb_hbm, o_ref,
             tab_ref, sem):
    b = pl.program_id(0)

    @pl.when(b == 0)
    def _():
        pltpu.make_async_copy(
            emb_hbm.at[pl.ds(0, EMB_ROWS), :],
            tab_ref.at[pl.ds(R, EMB_ROWS), :],
            sem,
        ).start()
        pltpu.make_async_copy(
            emb_hbm.at[pl.ds(0, EMB_ROWS), :],
            tab_ref.at[pl.ds(R, EMB_ROWS), :],
            sem,
        ).wait()

    base = b * R
    s0 = seg_first_ref[b]

    off0 = base - starts_ref[s0] + R
    q0 = pl.multiple_of((off0 // 8) * 8, 8)

    def store_piece0(k):
        def f():
            big = tab_ref[pl.ds(q0, R + 8), :]
            o_ref[:, :] = big[k:k + R, :] + x_ref[:, :]
        return f

    lax.switch(off0 - q0, [store_piece0(k) for k in range(8)])

    riota = lax.broadcasted_iota(jnp.int32, (R, 1), 0)

    def piece(j, carry):
        st = starts_ref[s0 + j]
        boundary = st - base
        offj = base - st + R
        qj = pl.multiple_of((offj // 8) * 8, 8)

        def store_piecej(k):
            def f():
                big = tab_ref[pl.ds(qj, R + 8), :]
                o_ref[:, :] = jnp.where(
                    riota >= boundary,
                    big[k:k + R, :] + x_ref[:, :],
                    o_ref[:, :])
            return f

        lax.switch(offj - qj, [store_piecej(k) for k in range(8)])
        return carry

    lax.fori_loop(1, n_extra_ref[b] + 1, piece, 0)


def _tc_call(x, emb, starts, seg_first, n_extra, interpret=False):
    total = x.shape[0]
    nb = (total + R - 1) // R
    grid_spec = pltpu.PrefetchScalarGridSpec(
        num_scalar_prefetch=3,
        grid=(nb,),
        in_specs=[
            pl.BlockSpec((R, DIM), lambda b, *_: (b, 0)),
            pl.BlockSpec(memory_space=pltpu.MemorySpace.HBM),
        ],
        out_specs=pl.BlockSpec((R, DIM), lambda b, *_: (b, 0)),
        scratch_shapes=[
            pltpu.VMEM((S_ROWS, DIM), jnp.float32),
            pltpu.SemaphoreType.DMA,
        ],
    )
    return pl.pallas_call(
        _tc_body,
        grid_spec=grid_spec,
        out_shape=jax.ShapeDtypeStruct((total, DIM), jnp.float32),
        compiler_params=pltpu.CompilerParams(
            dimension_semantics=("arbitrary",)),
        interpret=interpret,
    )(starts, seg_first, n_extra, x, emb)


def kernel(x, emb, sizes):
    total = x.shape[0]
    nb = (total + R - 1) // R
    sizes = sizes.astype(jnp.int32)
    csum = jnp.cumsum(sizes)
    starts = (csum - sizes).astype(jnp.int32)
    bstart = jnp.arange(nb, dtype=jnp.int32) * R
    seg_first = jnp.searchsorted(csum, bstart, side="right").astype(jnp.int32)
    last_row = jnp.minimum(bstart + (R - 1), total - 1)
    seg_last = jnp.searchsorted(csum, last_row, side="right").astype(jnp.int32)
    n_extra = seg_last - seg_first
    return _tc_call(x, emb, starts, seg_first, n_extra)


# speedup vs baseline: 3.0975x; 1.1316x over previous
import jax
import jax.numpy as jnp
from jax import lax
from jax.experimental import pallas as pl
from jax.experimental.pallas import tpu as pltpu

DIM = 1024
EMB_ROWS = 4096
R = 1024
S_ROWS = 2 * R + EMB_ROWS + 8


def _tc_body(starts_ref, seg_first_ref, n_extra_ref, x_ref, emb_hbm, o_ref,
             tab_ref, sem):
    b = pl.program_id(0)

    @pl.when(b == 0)
    def _():
        pltpu.make_async_copy(
            emb_hbm.at[pl.ds(0, EMB_ROWS), :],
            tab_ref.at[pl.ds(R, EMB_ROWS), :],
            sem,
        ).start()
        pltpu.make_async_copy(
            emb_hbm.at[pl.ds(0, EMB_ROWS), :],
            tab_ref.at[pl.ds(R, EMB_ROWS), :],
            sem,
        ).wait()

    base = b * R
    s0 = seg_first_ref[b]

    off0 = base - starts_ref[s0] + R
    q0 = pl.multiple_of((off0 // 8) * 8, 8)

    def store_piece0(k):
        def f():
            big = tab_ref[pl.ds(q0, R + 8), :]
            o_ref[:, :] = big[k:k + R, :] + x_ref[:, :]
        return f

    lax.switch(off0 - q0, [store_piece0(k) for k in range(8)])

    riota = lax.broadcasted_iota(jnp.int32, (R, 1), 0)

    def piece(j, carry):
        st = starts_ref[s0 + j]
        boundary = st - base
        offj = base - st + R
        qj = pl.multiple_of((offj // 8) * 8, 8)

        def store_piecej(k):
            def f():
                big = tab_ref[pl.ds(qj, R + 8), :]
                o_ref[:, :] = jnp.where(
                    riota >= boundary,
                    big[k:k + R, :] + x_ref[:, :],
                    o_ref[:, :])
            return f

        lax.switch(offj - qj, [store_piecej(k) for k in range(8)])
        return carry

    lax.fori_loop(1, n_extra_ref[b] + 1, piece, 0)


def _tc_call(x, emb, starts, seg_first, n_extra, interpret=False):
    total = x.shape[0]
    nb = (total + R - 1) // R
    grid_spec = pltpu.PrefetchScalarGridSpec(
        num_scalar_prefetch=3,
        grid=(nb,),
        in_specs=[
            pl.BlockSpec((R, DIM), lambda b, *_: (b, 0)),
            pl.BlockSpec(memory_space=pltpu.MemorySpace.HBM),
        ],
        out_specs=pl.BlockSpec((R, DIM), lambda b, *_: (b, 0)),
        scratch_shapes=[
            pltpu.VMEM((S_ROWS, DIM), jnp.float32),
            pltpu.SemaphoreType.DMA,
        ],
    )
    return pl.pallas_call(
        _tc_body,
        grid_spec=grid_spec,
        out_shape=jax.ShapeDtypeStruct((total, DIM), jnp.float32),
        compiler_params=pltpu.CompilerParams(
            dimension_semantics=("arbitrary",)),
        interpret=interpret,
    )(starts, seg_first, n_extra, x, emb)


def kernel(x, emb, sizes):
    total = x.shape[0]
    nb = (total + R - 1) // R
    sizes = sizes.astype(jnp.int32)
    csum = jnp.cumsum(sizes)
    starts = (csum - sizes).astype(jnp.int32)
    bstart = jnp.arange(nb, dtype=jnp.int32) * R
    seg_first = jnp.searchsorted(csum, bstart, side="right").astype(jnp.int32)
    last_row = jnp.minimum(bstart + (R - 1), total - 1)
    seg_last = jnp.searchsorted(csum, last_row, side="right").astype(jnp.int32)
    n_extra = seg_last - seg_first
    return _tc_call(x, emb, starts, seg_first, n_extra)


# speedup vs baseline: 3.1806x; 1.0268x over previous
import jax
import jax.numpy as jnp
from jax import lax
from jax.experimental import pallas as pl
from jax.experimental.pallas import tpu as pltpu

DIM = 1024
EMB_ROWS = 4096
R = 1536
S_ROWS = 2 * R + EMB_ROWS + 8


def _tc_body(starts_ref, seg_first_ref, n_extra_ref, x_ref, emb_hbm, o_ref,
             tab_ref, sem):
    b = pl.program_id(0)

    @pl.when(b == 0)
    def _():
        pltpu.make_async_copy(
            emb_hbm.at[pl.ds(0, EMB_ROWS), :],
            tab_ref.at[pl.ds(R, EMB_ROWS), :],
            sem,
        ).start()
        pltpu.make_async_copy(
            emb_hbm.at[pl.ds(0, EMB_ROWS), :],
            tab_ref.at[pl.ds(R, EMB_ROWS), :],
            sem,
        ).wait()

    base = b * R
    s0 = seg_first_ref[b]

    off0 = base - starts_ref[s0] + R
    q0 = pl.multiple_of((off0 // 8) * 8, 8)

    def store_piece0(k):
        def f():
            big = tab_ref[pl.ds(q0, R + 8), :]
            o_ref[:, :] = big[k:k + R, :] + x_ref[:, :]
        return f

    lax.switch(off0 - q0, [store_piece0(k) for k in range(8)])

    riota = lax.broadcasted_iota(jnp.int32, (R, 1), 0)

    def piece(j, carry):
        st = starts_ref[s0 + j]
        boundary = st - base
        offj = base - st + R
        qj = pl.multiple_of((offj // 8) * 8, 8)

        def store_piecej(k):
            def f():
                big = tab_ref[pl.ds(qj, R + 8), :]
                o_ref[:, :] = jnp.where(
                    riota >= boundary,
                    big[k:k + R, :] + x_ref[:, :],
                    o_ref[:, :])
            return f

        lax.switch(offj - qj, [store_piecej(k) for k in range(8)])
        return carry

    lax.fori_loop(1, n_extra_ref[b] + 1, piece, 0)


def _tc_call(x, emb, starts, seg_first, n_extra, interpret=False):
    total = x.shape[0]
    nb = (total + R - 1) // R
    grid_spec = pltpu.PrefetchScalarGridSpec(
        num_scalar_prefetch=3,
        grid=(nb,),
        in_specs=[
            pl.BlockSpec((R, DIM), lambda b, *_: (b, 0)),
            pl.BlockSpec(memory_space=pltpu.MemorySpace.HBM),
        ],
        out_specs=pl.BlockSpec((R, DIM), lambda b, *_: (b, 0)),
        scratch_shapes=[
            pltpu.VMEM((S_ROWS, DIM), jnp.float32),
            pltpu.SemaphoreType.DMA,
        ],
    )
    return pl.pallas_call(
        _tc_body,
        grid_spec=grid_spec,
        out_shape=jax.ShapeDtypeStruct((total, DIM), jnp.float32),
        compiler_params=pltpu.CompilerParams(
            dimension_semantics=("arbitrary",)),
        interpret=interpret,
    )(starts, seg_first, n_extra, x, emb)


def kernel(x, emb, sizes):
    total = x.shape[0]
    nb = (total + R - 1) // R
    sizes = sizes.astype(jnp.int32)
    csum = jnp.cumsum(sizes)
    starts = (csum - sizes).astype(jnp.int32)
    bstart = jnp.arange(nb, dtype=jnp.int32) * R
    seg_first = jnp.searchsorted(csum, bstart, side="right").astype(jnp.int32)
    last_row = jnp.minimum(bstart + (R - 1), total - 1)
    seg_last = jnp.searchsorted(csum, last_row, side="right").astype(jnp.int32)
    n_extra = seg_last - seg_first
    return _tc_call(x, emb, starts, seg_first, n_extra)
